# baseline (speedup 1.0000x reference)
import functools

import jax
import jax.numpy as jnp
from jax import lax
from jax.experimental import pallas as pl
from jax.experimental.pallas import tpu as pltpu
from jax.experimental.pallas import tpu_sc as plsc

N_NODES = 100000
D_FEAT = 128
DIMS = 32
B = 2048
S = 16

NW = 32
BPW = B // NW
SEGS = B * S
SEG_PER_W = SEGS // NW
CHUNK_SEGS = 8
CHUNKS = SEG_PER_W // CHUNK_SEGS
ROWS_PER_CHUNK = CHUNK_SEGS * S

PACK = 4


QROWS = N_NODES // PACK


def _t_body(f0, f1, f2, f3, w_ref, b_ref, o_ref):
    w = w_ref[...]
    bv = b_ref[0:1, :]
    cols = []
    for f in (f0, f1, f2, f3):
        x = jnp.dot(f[...], w, preferred_element_type=jnp.float32)
        cols.append(jnp.maximum(x + bv, 0.0))
    o_ref[...] = jnp.concatenate(cols, axis=1)


def _compute_t(features, W1, b1):
    blk = 1000
    nblk = QROWS // blk
    b1b = jnp.broadcast_to(b1.reshape(1, DIMS), (8, DIMS))
    fspec = [
        pl.BlockSpec((blk, D_FEAT),
                     functools.partial(lambda p, i: (i + p * nblk, 0), p))
        for p in range(PACK)
    ]
    t4 = pl.pallas_call(
        _t_body,
        grid=(nblk,),
        in_specs=fspec + [
            pl.BlockSpec((D_FEAT, DIMS), lambda i: (0, 0)),
            pl.BlockSpec((8, DIMS), lambda i: (0, 0)),
        ],
        out_specs=pl.BlockSpec((blk, PACK * DIMS), lambda i: (i, 0)),
        out_shape=jax.ShapeDtypeStruct((QROWS, PACK * DIMS), jnp.float32),
    )(features, features, features, features, W1, b1b)
    return t4.reshape(N_NODES, DIMS)


def _sc_body(t_hbm, s2i_hbm, s1i_hbm, bn_hbm, sum2_o, ts1_o, tb_o,
             ids2, ids1, idxb, idx2, buf_a, buf_b, buf_c, buf_d,
             outb, outc, sem_a, sem_b):
    wid = lax.axis_index("s") * 2 + lax.axis_index("c")
    b0 = wid * BPW
    iota16 = lax.iota(jnp.int32, 16)

    pltpu.sync_copy(s2i_hbm.at[:, :, pl.ds(b0, BPW)], ids2)
    pltpu.sync_copy(s1i_hbm.at[:, pl.ds(b0, BPW)], ids1)
    pltpu.sync_copy(bn_hbm.at[pl.ds(b0, BPW)], idxb)

    one = jnp.full((16,), 1, jnp.int32)
    zero = jnp.full((16,), 0, jnp.int32)

    def remap(v):
        q = (jnp.where(v >= QROWS, one, zero)
             + jnp.where(v >= 2 * QROWS, one, zero)
             + jnp.where(v >= 3 * QROWS, one, zero))
        return v * PACK - q * (PACK * QROWS - 1)

    for s in range(S):
        for h in range(BPW // 16):
            ids1[s, pl.ds(h * 16, 16)] = remap(ids1[s, pl.ds(h * 16, 16)])
    for h in range(BPW // 16):
        idxb[pl.ds(h * 16, 16)] = remap(idxb[pl.ds(h * 16, 16)])

    def build_row(c):
        raw = []
        for k in range(CHUNK_SEGS):
            g = c * CHUNK_SEGS + k
            s_out = g // BPW
            b_loc = g % BPW
            sv = jnp.full((16,), s_out, jnp.int32)
            bv = jnp.full((16,), b_loc, jnp.int32)
            raw.append(plsc.load_gather(ids2, [iota16, sv, bv]))
        for k in range(CHUNK_SEGS):
            idx2[c, pl.ds(k * 16, 16)] = remap(raw[k])

    def process(buf, c):
        s_out = lax.shift_right_logical(c, 3)
        cm = lax.bitwise_and(c, 7)
        for k in range(CHUNK_SEGS):
            r = k * S
            a0 = buf[r, pl.ds(0, 16)]
            a1 = buf[r, pl.ds(16, 16)]
            for j in range(1, S):
                a0 = a0 + buf[r + j, pl.ds(0, 16)]
                a1 = a1 + buf[r + j, pl.ds(16, 16)]
            row = cm * 2 + (k // 4)
            outb[s_out, row, pl.ds((k % 4) * 32, 16)] = a0
            outb[s_out, row, pl.ds((k % 4) * 32 + 16, 16)] = a1

    with jax.named_scope("scp_prelude"):
        for c in range(3):
            build_row(c)
    pltpu.async_copy(t_hbm.at[idx2.at[0]], buf_a, sem_a)

    def body(c2, carry):
        c = c2 * 2
        pltpu.async_copy(t_hbm.at[idx2.at[c + 1]], buf_b, sem_b)

        @pl.when(c2 < CHUNKS // 2 - 1)
        def _():
            build_row(c + 3)

        @pl.when(c2 < CHUNKS // 2 - 2)
        def _():
            build_row(c + 4)

        pltpu.make_async_copy(t_hbm.at[idx2.at[c]], buf_a, sem_a).wait()
        process(buf_a, c)

        @pl.when(c2 < CHUNKS // 2 - 1)
        def _():
            pltpu.async_copy(t_hbm.at[idx2.at[c + 2]], buf_a, sem_a)

        pltpu.make_async_copy(t_hbm.at[idx2.at[c + 1]], buf_b, sem_b).wait()
        process(buf_b, c + 1)
        return carry

    with jax.named_scope("scp_taskA"):
        lax.fori_loop(0, CHUNKS // 2, body, 0)
    with jax.named_scope("scp_sum2_flush"):
        pltpu.sync_copy(outb,
                        sum2_o.at[:, pl.ds(wid * (BPW // PACK), BPW // PACK)])

    def repack3(buf, s):
        def rp(m, carry):
            for sub in range(PACK):
                outb[s, m, pl.ds(sub * 32, 16)] = buf[m * 4 + sub, pl.ds(0, 16)]
                outb[s, m, pl.ds(sub * 32 + 16, 16)] = buf[m * 4 + sub,
                                                           pl.ds(16, 16)]
            return carry

        lax.fori_loop(0, BPW // PACK, rp, 0)

    pltpu.async_copy(t_hbm.at[ids1.at[0]], buf_c, sem_a)

    def ts1_body(i2, carry):
        s = i2 * 2
        pltpu.async_copy(t_hbm.at[ids1.at[s + 1]], buf_d, sem_b)
        pltpu.make_async_copy(t_hbm.at[ids1.at[s]], buf_c, sem_a).wait()
        repack3(buf_c, s)

        @pl.when(i2 < S // 2 - 1)
        def _():
            pltpu.async_copy(t_hbm.at[ids1.at[s + 2]], buf_c, sem_a)

        pltpu.make_async_copy(t_hbm.at[ids1.at[s + 1]], buf_d, sem_b).wait()
        repack3(buf_d, s + 1)
        return carry

    with jax.named_scope("scp_ts1"):
        lax.fori_loop(0, S // 2, ts1_body, 0)
        pltpu.sync_copy(outb,
                        ts1_o.at[:, pl.ds(wid * (BPW // PACK), BPW // PACK)])

    pltpu.async_copy(t_hbm.at[idxb], buf_c, sem_a).wait()

    def rp_tb(m, carry):
        for sub in range(PACK):
            outc[m, pl.ds(sub * 32, 16)] = buf_c[m * 4 + sub, pl.ds(0, 16)]
            outc[m, pl.ds(sub * 32 + 16, 16)] = buf_c[m * 4 + sub,
                                                      pl.ds(16, 16)]
        return carry

    lax.fori_loop(0, BPW // PACK, rp_tb, 0)
    pltpu.sync_copy(outc, tb_o.at[pl.ds(wid * (BPW // PACK), BPW // PACK)])


_sc_gather = functools.partial(
    pl.kernel,
    out_type=(
        jax.ShapeDtypeStruct((S, B // PACK, PACK * DIMS), jnp.float32),
        jax.ShapeDtypeStruct((S, B // PACK, PACK * DIMS), jnp.float32),
        jax.ShapeDtypeStruct((B // PACK, PACK * DIMS), jnp.float32),
    ),
    mesh=plsc.VectorSubcoreMesh(core_axis_name="c", subcore_axis_name="s"),
    compiler_params=pltpu.CompilerParams(use_tc_tiling_on_sc=False,
                                         needs_layout_passes=False),
    scratch_types=[
        pltpu.VMEM((S, S, BPW), jnp.int32),
        pltpu.VMEM((S, BPW), jnp.int32),
        pltpu.VMEM((BPW,), jnp.int32),
        pltpu.VMEM((CHUNKS, ROWS_PER_CHUNK), jnp.int32),
        pltpu.VMEM((ROWS_PER_CHUNK, DIMS), jnp.float32),
        pltpu.VMEM((ROWS_PER_CHUNK, DIMS), jnp.float32),
        pltpu.VMEM((BPW, DIMS), jnp.float32),
        pltpu.VMEM((BPW, DIMS), jnp.float32),
        pltpu.VMEM((S, BPW // PACK, PACK * DIMS), jnp.float32),
        pltpu.VMEM((BPW // PACK, PACK * DIMS), jnp.float32),
        pltpu.SemaphoreType.DMA,
        pltpu.SemaphoreType.DMA,
    ],
)(_sc_body)


def _s3_body(s2_ref, t1_ref, tb_ref, w2_ref, b2_ref, o_ref):
    w2 = w2_ref[...]
    b2v = b2_ref[0:1, :]
    acc_l = jnp.zeros(tb_ref.shape, jnp.float32)
    acc_s = jnp.zeros(tb_ref.shape, jnp.float32)
    for s in range(S):
        t1 = t1_ref[s]
        an1 = (s2_ref[s] + t1) * (1.0 / 17.0)
        h = jnp.maximum(jnp.dot(an1, w2, preferred_element_type=jnp.float32) + b2v, 0.0)
        acc_l = acc_l + h
        acc_s = acc_s + t1
    an0 = (acc_s + tb_ref[...]) * (1.0 / 17.0)
    h0 = jnp.maximum(jnp.dot(an0, w2, preferred_element_type=jnp.float32) + b2v, 0.0)
    o_ref[...] = (acc_l + h0) * (1.0 / 17.0)


def _stage3(sum2, ts1, tb, W2, b2):
    blk = 64
    bp = B // PACK
    w2d = jnp.kron(jnp.eye(PACK, dtype=jnp.float32), W2)
    b2b = jnp.broadcast_to(jnp.tile(b2, PACK).reshape(1, PACK * DIMS),
                           (8, PACK * DIMS))
    out = pl.pallas_call(
        _s3_body,
        grid=(bp // blk,),
        in_specs=[
            pl.BlockSpec((S, blk, PACK * DIMS), lambda i: (0, i, 0)),
            pl.BlockSpec((S, blk, PACK * DIMS), lambda i: (0, i, 0)),
            pl.BlockSpec((blk, PACK * DIMS), lambda i: (i, 0)),
            pl.BlockSpec((PACK * DIMS, PACK * DIMS), lambda i: (0, 0)),
            pl.BlockSpec((8, PACK * DIMS), lambda i: (0, 0)),
        ],
        out_specs=pl.BlockSpec((blk, PACK * DIMS), lambda i: (i, 0)),
        out_shape=jax.ShapeDtypeStruct((bp, PACK * DIMS), jnp.float32),
    )(sum2, ts1, tb, w2d, b2b)
    return out.reshape(B, DIMS)


def kernel(features, batch_nodes, s1_neighs, s2_neighs, W1, b1, W2, b2):
    T = _compute_t(features, W1, b1)

    s2i = jnp.transpose(s2_neighs.astype(jnp.int32), (1, 2, 0))
    s1i = jnp.transpose(s1_neighs.astype(jnp.int32), (1, 0))
    bnf = batch_nodes.astype(jnp.int32)

    sum2, ts1, tb = _sc_gather(T, s2i, s1i, bnf)
    return _stage3(sum2, ts1, tb, W2, b2)


# speedup vs baseline: 1.0516x; 1.0516x over previous
import functools

import jax
import jax.numpy as jnp
from jax import lax
from jax.experimental import pallas as pl
from jax.experimental.pallas import tpu as pltpu
from jax.experimental.pallas import tpu_sc as plsc

N_NODES = 100000
D_FEAT = 128
DIMS = 32
B = 2048
S = 16

NW = 32
BPW = B // NW
SEGS = B * S
SEG_PER_W = SEGS // NW
CHUNK_SEGS = 8
CHUNKS = SEG_PER_W // CHUNK_SEGS
ROWS_PER_CHUNK = CHUNK_SEGS * S

PACK = 4

NQ = 8
QROWS = 12504


def _t_body(*refs):
    fs, w_ref, b_ref, o_ref = refs[:NQ], refs[NQ], refs[NQ + 1], refs[NQ + 2]
    w = w_ref[...]
    bv = b_ref[0:1, :]
    cols = []
    for f in fs:
        x = jnp.dot(f[...], w, preferred_element_type=jnp.float32)
        y = jnp.maximum(x + bv, 0.0)
        lo = lax.bitcast_convert_type(y[:, :16].astype(jnp.bfloat16),
                                      jnp.uint16).astype(jnp.uint32)
        hi = lax.bitcast_convert_type(y[:, 16:].astype(jnp.bfloat16),
                                      jnp.uint16).astype(jnp.uint32)
        u = jnp.bitwise_or(lo, jnp.left_shift(hi, 16))
        cols.append(lax.bitcast_convert_type(u, jnp.float32))
    o_ref[...] = jnp.concatenate(cols, axis=1)


def _compute_t(features, W1, b1):
    blk = QROWS // 3
    b1b = jnp.broadcast_to(b1.reshape(1, DIMS), (8, DIMS))
    fspec = [
        pl.BlockSpec((blk, D_FEAT),
                     functools.partial(lambda p, i: (i + p * 3, 0), p))
        for p in range(NQ)
    ]
    t8 = pl.pallas_call(
        _t_body,
        grid=(3,),
        in_specs=fspec + [
            pl.BlockSpec((D_FEAT, DIMS), lambda i: (0, 0)),
            pl.BlockSpec((8, DIMS), lambda i: (0, 0)),
        ],
        out_specs=pl.BlockSpec((blk, NQ * 16), lambda i: (i, 0)),
        out_shape=jax.ShapeDtypeStruct((QROWS, NQ * 16), jnp.float32),
    )(*([features] * NQ), W1, b1b)
    return t8.reshape(QROWS * NQ, 16)


def _sc_body(t_hbm, s2i_hbm, s1i_hbm, bn_hbm, sum2_o, ts1_o, tb_o,
             ids2, ids1, idxb, idx2, buf_a, buf_b, buf_c, buf_d,
             outb, outc, sem_a, sem_b):
    wid = lax.axis_index("s") * 2 + lax.axis_index("c")
    b0 = wid * BPW
    iota16 = lax.iota(jnp.int32, 16)

    pltpu.sync_copy(s2i_hbm.at[:, :, pl.ds(b0, BPW)], ids2)
    pltpu.sync_copy(s1i_hbm.at[:, pl.ds(b0, BPW)], ids1)
    pltpu.sync_copy(bn_hbm.at[pl.ds(b0, BPW)], idxb)

    one = jnp.full((16,), 1, jnp.int32)
    zero = jnp.full((16,), 0, jnp.int32)

    def remap(v):
        q = zero
        for k in range(1, NQ):
            q = q + jnp.where(v >= k * QROWS, one, zero)
        return v * NQ - q * (NQ * QROWS - 1)

    for s in range(S):
        for h in range(BPW // 16):
            ids1[s, pl.ds(h * 16, 16)] = remap(ids1[s, pl.ds(h * 16, 16)])
    for h in range(BPW // 16):
        idxb[pl.ds(h * 16, 16)] = remap(idxb[pl.ds(h * 16, 16)])

    def build_row(c):
        raw = []
        for k in range(CHUNK_SEGS):
            g = c * CHUNK_SEGS + k
            s_out = g // BPW
            b_loc = g % BPW
            sv = jnp.full((16,), s_out, jnp.int32)
            bv = jnp.full((16,), b_loc, jnp.int32)
            raw.append(plsc.load_gather(ids2, [iota16, sv, bv]))
        for k in range(CHUNK_SEGS):
            idx2[c, pl.ds(k * 16, 16)] = remap(raw[k])

    def unpk(row16):
        return plsc.unpack(plsc.bitcast(row16, jnp.bfloat16),
                           format=plsc.PackFormat.INTERLEAVED,
                           preferred_element_type=jnp.float32)

    def process(buf, c):
        s_out = lax.shift_right_logical(c, 3)
        cm = lax.bitwise_and(c, 7)
        for k in range(CHUNK_SEGS):
            r = k * S
            a0, a1 = unpk(buf[r])
            for j in range(1, S):
                u0, u1 = unpk(buf[r + j])
                a0 = a0 + u0
                a1 = a1 + u1
            row = cm * 2 + (k // 4)
            outb[s_out, row, pl.ds((k % 4) * 32, 16)] = a0
            outb[s_out, row, pl.ds((k % 4) * 32 + 16, 16)] = a1

    with jax.named_scope("scp_prelude"):
        for c in range(3):
            build_row(c)
    pltpu.async_copy(t_hbm.at[idx2.at[0]], buf_a, sem_a)

    def body(c2, carry):
        c = c2 * 2
        pltpu.async_copy(t_hbm.at[idx2.at[c + 1]], buf_b, sem_b)

        @pl.when(c2 < CHUNKS // 2 - 1)
        def _():
            build_row(c + 3)

        @pl.when(c2 < CHUNKS // 2 - 2)
        def _():
            build_row(c + 4)

        pltpu.make_async_copy(t_hbm.at[idx2.at[c]], buf_a, sem_a).wait()
        process(buf_a, c)

        @pl.when(c2 < CHUNKS // 2 - 1)
        def _():
            pltpu.async_copy(t_hbm.at[idx2.at[c + 2]], buf_a, sem_a)

        pltpu.make_async_copy(t_hbm.at[idx2.at[c + 1]], buf_b, sem_b).wait()
        process(buf_b, c + 1)
        return carry

    with jax.named_scope("scp_taskA"):
        lax.fori_loop(0, CHUNKS // 2, body, 0)
    with jax.named_scope("scp_sum2_flush"):
        pltpu.sync_copy(outb,
                        sum2_o.at[:, pl.ds(wid * (BPW // PACK), BPW // PACK)])

    def repack3(buf, s):
        def rp(m, carry):
            for sub in range(PACK):
                u0, u1 = unpk(buf[m * 4 + sub])
                outb[s, m, pl.ds(sub * 32, 16)] = u0
                outb[s, m, pl.ds(sub * 32 + 16, 16)] = u1
            return carry

        lax.fori_loop(0, BPW // PACK, rp, 0)

    pltpu.async_copy(t_hbm.at[ids1.at[0]], buf_c, sem_a)

    def ts1_body(i2, carry):
        s = i2 * 2
        pltpu.async_copy(t_hbm.at[ids1.at[s + 1]], buf_d, sem_b)
        pltpu.make_async_copy(t_hbm.at[ids1.at[s]], buf_c, sem_a).wait()
        repack3(buf_c, s)

        @pl.when(i2 < S // 2 - 1)
        def _():
            pltpu.async_copy(t_hbm.at[ids1.at[s + 2]], buf_c, sem_a)

        pltpu.make_async_copy(t_hbm.at[ids1.at[s + 1]], buf_d, sem_b).wait()
        repack3(buf_d, s + 1)
        return carry

    with jax.named_scope("scp_ts1"):
        lax.fori_loop(0, S // 2, ts1_body, 0)
        pltpu.sync_copy(outb,
                        ts1_o.at[:, pl.ds(wid * (BPW // PACK), BPW // PACK)])

    pltpu.async_copy(t_hbm.at[idxb], buf_c, sem_a).wait()

    def rp_tb(m, carry):
        for sub in range(PACK):
            u0, u1 = unpk(buf_c[m * 4 + sub])
            outc[m, pl.ds(sub * 32, 16)] = u0
            outc[m, pl.ds(sub * 32 + 16, 16)] = u1
        return carry

    lax.fori_loop(0, BPW // PACK, rp_tb, 0)
    pltpu.sync_copy(outc, tb_o.at[pl.ds(wid * (BPW // PACK), BPW // PACK)])


_sc_gather = functools.partial(
    pl.kernel,
    out_type=(
        jax.ShapeDtypeStruct((S, B // PACK, PACK * DIMS), jnp.float32),
        jax.ShapeDtypeStruct((S, B // PACK, PACK * DIMS), jnp.float32),
        jax.ShapeDtypeStruct((B // PACK, PACK * DIMS), jnp.float32),
    ),
    mesh=plsc.VectorSubcoreMesh(core_axis_name="c", subcore_axis_name="s"),
    compiler_params=pltpu.CompilerParams(use_tc_tiling_on_sc=False,
                                         needs_layout_passes=False),
    scratch_types=[
        pltpu.VMEM((S, S, BPW), jnp.int32),
        pltpu.VMEM((S, BPW), jnp.int32),
        pltpu.VMEM((BPW,), jnp.int32),
        pltpu.VMEM((CHUNKS, ROWS_PER_CHUNK), jnp.int32),
        pltpu.VMEM((ROWS_PER_CHUNK, 16), jnp.float32),
        pltpu.VMEM((ROWS_PER_CHUNK, 16), jnp.float32),
        pltpu.VMEM((BPW, 16), jnp.float32),
        pltpu.VMEM((BPW, 16), jnp.float32),
        pltpu.VMEM((S, BPW // PACK, PACK * DIMS), jnp.float32),
        pltpu.VMEM((BPW // PACK, PACK * DIMS), jnp.float32),
        pltpu.SemaphoreType.DMA,
        pltpu.SemaphoreType.DMA,
    ],
)(_sc_body)


def _s3_body(s2_ref, t1_ref, tb_ref, w2_ref, b2_ref, o_ref):
    w2 = w2_ref[...]
    b2v = b2_ref[0:1, :]
    acc_l = jnp.zeros(tb_ref.shape, jnp.float32)
    acc_s = jnp.zeros(tb_ref.shape, jnp.float32)
    for s in range(S):
        t1 = t1_ref[s]
        an1 = (s2_ref[s] + t1) * (1.0 / 17.0)
        h = jnp.maximum(jnp.dot(an1, w2, preferred_element_type=jnp.float32) + b2v, 0.0)
        acc_l = acc_l + h
        acc_s = acc_s + t1
    an0 = (acc_s + tb_ref[...]) * (1.0 / 17.0)
    h0 = jnp.maximum(jnp.dot(an0, w2, preferred_element_type=jnp.float32) + b2v, 0.0)
    o_ref[...] = (acc_l + h0) * (1.0 / 17.0)


def _stage3(sum2, ts1, tb, W2, b2):
    blk = 64
    bp = B // PACK
    w2d = jnp.kron(jnp.eye(PACK, dtype=jnp.float32), W2)
    b2b = jnp.broadcast_to(jnp.tile(b2, PACK).reshape(1, PACK * DIMS),
                           (8, PACK * DIMS))
    out = pl.pallas_call(
        _s3_body,
        grid=(bp // blk,),
        in_specs=[
            pl.BlockSpec((S, blk, PACK * DIMS), lambda i: (0, i, 0)),
            pl.BlockSpec((S, blk, PACK * DIMS), lambda i: (0, i, 0)),
            pl.BlockSpec((blk, PACK * DIMS), lambda i: (i, 0)),
            pl.BlockSpec((PACK * DIMS, PACK * DIMS), lambda i: (0, 0)),
            pl.BlockSpec((8, PACK * DIMS), lambda i: (0, 0)),
        ],
        out_specs=pl.BlockSpec((blk, PACK * DIMS), lambda i: (i, 0)),
        out_shape=jax.ShapeDtypeStruct((bp, PACK * DIMS), jnp.float32),
    )(sum2, ts1, tb, w2d, b2b)
    return out.reshape(B, DIMS)


def kernel(features, batch_nodes, s1_neighs, s2_neighs, W1, b1, W2, b2):
    T = _compute_t(features, W1, b1)

    s2i = jnp.transpose(s2_neighs.astype(jnp.int32), (1, 2, 0))
    s1i = jnp.transpose(s1_neighs.astype(jnp.int32), (1, 0))
    bnf = batch_nodes.astype(jnp.int32)

    sum2, ts1, tb = _sc_gather(T, s2i, s1i, bnf)
    return _stage3(sum2, ts1, tb, W2, b2)
